# Initial kernel scaffold; baseline (speedup 1.0000x reference)
import jax
import jax.numpy as jnp
from jax.experimental import pallas as pl


def kernel(token_ids, weight):
    raise NotImplementedError("write your pallas kernel here")


# speedup vs baseline: 1.7614x; 1.7614x over previous
import functools

import jax
import jax.numpy as jnp
from jax import lax
from jax.experimental import pallas as pl
from jax.experimental.pallas import tpu as pltpu
from jax.experimental.pallas import tpu_sc as plsc

NUM_CORES = 2
NUM_SUBCORES = 16
NUM_WORKERS = NUM_CORES * NUM_SUBCORES

CHUNK = 128


@functools.partial(jax.jit, static_argnames=("b", "d"))
def _embed_lookup(idx_flat, weight, *, b, d):
    b_per_w = b // NUM_WORKERS
    n_chunks = b_per_w // CHUNK
    mesh = plsc.VectorSubcoreMesh(
        core_axis_name="c", subcore_axis_name="s",
        num_cores=NUM_CORES, num_subcores=NUM_SUBCORES,
    )

    @functools.partial(
        pl.kernel,
        mesh=mesh,
        out_type=jax.ShapeDtypeStruct((b, d), jnp.float32),
        scratch_types=[
            pltpu.VMEM((b_per_w,), jnp.int32),
            pltpu.VMEM((2, CHUNK, d), jnp.float32),
            pltpu.SemaphoreType.DMA,
            pltpu.SemaphoreType.DMA,
        ],
        compiler_params=pltpu.CompilerParams(use_tc_tiling_on_sc=False),
    )
    def run(idx_hbm, table_hbm, out_hbm, idx_v, rows_v, gsem, osem):
        wid = lax.axis_index("s") * NUM_CORES + lax.axis_index("c")
        base = wid * b_per_w
        pltpu.sync_copy(idx_hbm.at[pl.ds(base, b_per_w)], idx_v)

        def gather(j, slot):
            return pltpu.async_copy(
                table_hbm.at[idx_v.at[pl.ds(j * CHUNK, CHUNK)]],
                rows_v.at[slot], gsem)

        def put(j, slot):
            return pltpu.async_copy(
                rows_v.at[slot], out_hbm.at[pl.ds(base + j * CHUNK, CHUNK)],
                osem)

        gather(0, 0).wait()

        def body(j, _):
            slot = lax.rem(j, 2)
            g = gather(j + 1, 1 - slot)
            put(j, slot).wait()
            g.wait()
            return 0

        lax.fori_loop(0, n_chunks - 1, body, 0, unroll=False)
        last = n_chunks - 1
        put(last, lax.rem(last, 2)).wait()

    return run(idx_flat, weight)


def kernel(token_ids, weight):
    s, t = token_ids.shape
    d = weight.shape[1]
    idx_flat = token_ids.reshape(s * t).astype(jnp.int32)
    out = _embed_lookup(idx_flat, weight, b=s * t, d=d)
    return out.reshape(s, t, d)


# speedup vs baseline: 1.8741x; 1.0640x over previous
import functools

import jax
import jax.numpy as jnp
from jax import lax
from jax.experimental import pallas as pl
from jax.experimental.pallas import tpu as pltpu
from jax.experimental.pallas import tpu_sc as plsc

NUM_CORES = 2
NUM_SUBCORES = 16
NUM_WORKERS = NUM_CORES * NUM_SUBCORES

CHUNK = 128
NBUF = 8


@functools.partial(jax.jit, static_argnames=("b", "d"))
def _embed_lookup(idx_flat, weight, *, b, d):
    b_per_w = b // NUM_WORKERS
    n_chunks = b_per_w // CHUNK
    mesh = plsc.VectorSubcoreMesh(
        core_axis_name="c", subcore_axis_name="s",
        num_cores=NUM_CORES, num_subcores=NUM_SUBCORES,
    )

    @functools.partial(
        pl.kernel,
        mesh=mesh,
        out_type=jax.ShapeDtypeStruct((b, d), jnp.float32),
        scratch_types=[
            pltpu.VMEM((b_per_w,), jnp.int32),
            pltpu.VMEM((NBUF, CHUNK, d), jnp.float32),
            pltpu.SemaphoreType.DMA,
            pltpu.SemaphoreType.DMA,
        ],
        compiler_params=pltpu.CompilerParams(use_tc_tiling_on_sc=False),
    )
    def run(idx_hbm, table_hbm, out_hbm, idx_v, rows_v, gsem, osem):
        wid = lax.axis_index("s") * NUM_CORES + lax.axis_index("c")
        base = wid * b_per_w
        pltpu.sync_copy(idx_hbm.at[pl.ds(base, b_per_w)], idx_v)

        def gather(j, slot):
            return pltpu.async_copy(
                table_hbm.at[idx_v.at[pl.ds(j * CHUNK, CHUNK)]],
                rows_v.at[slot], gsem)

        def put(j, slot):
            return pltpu.async_copy(
                rows_v.at[slot], out_hbm.at[pl.ds(base + j * CHUNK, CHUNK)],
                osem)

        def drain_gather(slot):
            pltpu.make_async_copy(
                table_hbm.at[idx_v.at[pl.ds(0, CHUNK)]], rows_v.at[slot],
                gsem).wait()

        for s in range(NBUF):
            gather(s, s)

        def body(j, _):
            slot = lax.rem(j, NBUF)
            drain_gather(slot)
            put(j, slot).wait()
            gather(j + NBUF, slot)
            return 0

        lax.fori_loop(0, n_chunks - NBUF, body, 0, unroll=False)

        for j in range(n_chunks - NBUF, n_chunks):
            slot = j % NBUF
            drain_gather(slot)
            put(j, slot).wait()

    return run(idx_flat, weight)


def kernel(token_ids, weight):
    s, t = token_ids.shape
    d = weight.shape[1]
    idx_flat = token_ids.reshape(s * t).astype(jnp.int32)
    out = _embed_lookup(idx_flat, weight, b=s * t, d=d)
    return out.reshape(s, t, d)


# speedup vs baseline: 1.8788x; 1.0025x over previous
import functools

import jax
import jax.numpy as jnp
from jax import lax
from jax.experimental import pallas as pl
from jax.experimental.pallas import tpu as pltpu
from jax.experimental.pallas import tpu_sc as plsc

NUM_CORES = 2
NUM_SUBCORES = 16
NUM_WORKERS = NUM_CORES * NUM_SUBCORES

CHUNK = 256
NBUF = 6
KLAG = 3


@functools.partial(jax.jit, static_argnames=("b", "d"))
def _embed_lookup(idx_flat, weight, *, b, d):
    b_per_w = b // NUM_WORKERS
    n_chunks = b_per_w // CHUNK
    mesh = plsc.VectorSubcoreMesh(
        core_axis_name="c", subcore_axis_name="s",
        num_cores=NUM_CORES, num_subcores=NUM_SUBCORES,
    )

    @functools.partial(
        pl.kernel,
        mesh=mesh,
        out_type=jax.ShapeDtypeStruct((b, d), jnp.float32),
        scratch_types=[
            pltpu.VMEM((b_per_w,), jnp.int32),
            pltpu.VMEM((NBUF, CHUNK, d), jnp.float32),
            pltpu.SemaphoreType.DMA,
            pltpu.SemaphoreType.DMA,
        ],
        compiler_params=pltpu.CompilerParams(use_tc_tiling_on_sc=False),
    )
    def run(idx_hbm, table_hbm, out_hbm, idx_v, rows_v, gsem, osem):
        wid = lax.axis_index("s") * NUM_CORES + lax.axis_index("c")
        base = wid * b_per_w
        pltpu.sync_copy(idx_hbm.at[pl.ds(base, b_per_w)], idx_v)

        def gather(j, slot):
            return pltpu.async_copy(
                table_hbm.at[idx_v.at[pl.ds(j * CHUNK, CHUNK)]],
                rows_v.at[slot], gsem)

        def put(j, slot):
            return pltpu.async_copy(
                rows_v.at[slot],
                out_hbm.at[pl.ds(base + j * CHUNK, CHUNK)], osem)

        def drain_gather(slot):
            pltpu.make_async_copy(
                table_hbm.at[idx_v.at[pl.ds(0, CHUNK)]], rows_v.at[slot],
                gsem).wait()

        def drain_put(slot):
            pltpu.make_async_copy(
                rows_v.at[slot], out_hbm.at[pl.ds(base, CHUNK)], osem).wait()

        for s in range(NBUF):
            gather(s, s)

        for j in range(KLAG):
            drain_gather(j % NBUF)
            put(j, j % NBUF)

        def body(j, _):
            slot = lax.rem(j, NBUF)
            drain_gather(slot)
            put(j, slot)
            old = lax.rem(j - KLAG, NBUF)
            drain_put(old)
            gather(j - KLAG + NBUF, old)
            return 0

        lax.fori_loop(KLAG, n_chunks - NBUF + KLAG, body, 0, unroll=False)

        for j in range(n_chunks - NBUF + KLAG, n_chunks):
            slot = j % NBUF
            drain_gather(slot)
            put(j, slot)
        for j in range(n_chunks - NBUF, n_chunks):
            drain_put(j % NBUF)

    return run(idx_flat, weight)


def kernel(token_ids, weight):
    s, t = token_ids.shape
    d = weight.shape[1]
    idx_flat = token_ids.reshape(s * t).astype(jnp.int32)
    out = _embed_lookup(idx_flat, weight, b=s * t, d=d)
    return out.reshape(s, t, d)
